# baseline (speedup 1.0000x reference)
import functools

import jax
import jax.numpy as jnp
from jax import lax
from jax.experimental import pallas as pl
from jax.experimental.pallas import tpu as pltpu
from jax.experimental.pallas import tpu_sc as plsc

N_POS = 200
D = 64
BATCH = 4096
SEQ = 200
NTOK = BATCH * SEQ
NW = 32
TOK_PER_W = NTOK // NW
CHUNK = 128
NCH = TOK_PER_W // CHUNK
NBUF = 8
UNROLL = 16
COMPUTE_ON = True


def _rsqrt(a):
    i = lax.bitcast_convert_type(a, jnp.int32)
    i = jnp.int32(0x5F3759DF) - (i >> 1)
    y = lax.bitcast_convert_type(i, jnp.float32)
    h = a * 0.5
    for _ in range(2):
        y = y * (1.5 - h * y * y)
    return y


_GATHER_DNUMS = lax.GatherDimensionNumbers(
    offset_dims=(), collapsed_slice_dims=(0,), start_index_map=(0,))


def _shuf(v, perm2d):
    return lax.gather(v, perm2d, _GATHER_DNUMS, slice_sizes=(1,),
                      mode=lax.GatherScatterMode.PROMISE_IN_BOUNDS)


def _lane_sum(v, perms):
    for perm2d in perms:
        v = v + _shuf(v, perm2d)
    return v


def _ln_row(rows_v, src_i, y_v, dst_i, pos_v, p, gvec, bvec, perms):
    x = [rows_v[src_i, pl.ds(16 * j, 16)] + pos_v[p, pl.ds(16 * j, 16)]
         for j in range(4)]
    s = (x[0] + x[1]) + (x[2] + x[3])
    q = (x[0] * x[0] + x[1] * x[1]) + (x[2] * x[2] + x[3] * x[3])
    mean = _lane_sum(s, perms) * (1.0 / D)
    ex2 = _lane_sum(q, perms) * (1.0 / D)
    var = ex2 - mean * mean
    r = _rsqrt(var + 1e-5)
    for j in range(4):
        y = (x[j] - mean) * r * gvec[j] + bvec[j]
        y_v[dst_i, pl.ds(16 * j, 16)] = y


def _sc_body(instr_hbm, table_hbm, pos_hbm, gamma_hbm, beta_hbm, out_hbm,
             idx_all, rows_all, y0, y1, pos_v, g_v, b_v,
             gsems, ysem0, ysem1):
    wid = lax.axis_index("s") * 2 + lax.axis_index("c")
    wrow0 = wid * (TOK_PER_W // 128)
    ybuf = (y0, y1)
    ysem = (ysem0, ysem1)

    pltpu.sync_copy(instr_hbm.at[pl.ds(wrow0, TOK_PER_W // 128)], idx_all)
    pltpu.sync_copy(pos_hbm, pos_v)
    pltpu.sync_copy(gamma_hbm, g_v)
    pltpu.sync_copy(beta_hbm, b_v)
    gvec = [g_v[pl.ds(16 * j, 16)] for j in range(4)]
    bvec = [b_v[pl.ds(16 * j, 16)] for j in range(4)]
    lanes = lax.iota(jnp.int32, 16)
    perms = [(lanes ^ k).reshape(16, 1) for k in (8, 4, 2, 1)]

    def gather_start(g, b):
        pltpu.make_async_copy(
            table_hbm.at[idx_all.at[g]],
            rows_all.at[pl.ds(b * CHUNK, CHUNK)], gsems.at[b]).start()

    def gather_wait(g, b):
        pltpu.make_async_copy(
            table_hbm.at[idx_all.at[g]],
            rows_all.at[pl.ds(b * CHUNK, CHUNK)], gsems.at[b]).wait()

    def out_copy(g, b):
        return pltpu.make_async_copy(
            ybuf[b], out_hbm.at[pl.ds(wrow0 * 128 + g * CHUNK, CHUNK)],
            ysem[b])

    for b in range(NBUF):
        gather_start(b, b)

    def lap_body(t, carry):
        for b in range(NBUF):
            gc = NBUF * t + b
            gather_wait(gc, b)
            p0 = (gc * CHUNK) % N_POS

            if COMPUTE_ON:
                yb = ybuf[b % 2]

                @pl.when(gc >= 2)
                def _(b=b, gc=gc):
                    out_copy(gc - 2, b % 2).wait()

                @plsc.parallel_loop(0, CHUNK, 1, unroll=UNROLL)
                def _row(i, b=b, p0=p0, yb=yb):
                    tt = p0 + i
                    p = jnp.where(tt >= N_POS, tt - N_POS, tt)
                    _ln_row(rows_all, b * CHUNK + i, yb, i,
                            pos_v, p, gvec, bvec, perms)
                out_copy(gc, b % 2).start()

            @pl.when(gc + NBUF < NCH)
            def _(b=b, gc=gc):
                gather_start(gc + NBUF, b)
        return carry

    lax.fori_loop(0, NCH // NBUF, lap_body, 0)
    if COMPUTE_ON:
        out_copy(NCH - 2, (NCH - 2) % 2).wait()
        out_copy(NCH - 1, (NCH - 1) % 2).wait()


@jax.jit
def _run(instr2d, emb_table, pos_table, ln_gamma, ln_beta):
    mesh = plsc.VectorSubcoreMesh(core_axis_name="c", subcore_axis_name="s")
    f = pl.kernel(
        _sc_body,
        mesh=mesh,
        out_type=jax.ShapeDtypeStruct((NTOK, D), jnp.float32),
        scratch_types=[
            pltpu.VMEM((TOK_PER_W // 128, 128), jnp.int32),
            pltpu.VMEM((NBUF * CHUNK, D), jnp.float32),
            pltpu.VMEM((CHUNK, D), jnp.float32),
            pltpu.VMEM((CHUNK, D), jnp.float32),
            pltpu.VMEM((N_POS, D), jnp.float32),
            pltpu.VMEM((D,), jnp.float32),
            pltpu.VMEM((D,), jnp.float32),
            pltpu.SemaphoreType.DMA((NBUF,)),
            pltpu.SemaphoreType.DMA,
            pltpu.SemaphoreType.DMA,
        ],
        compiler_params=pltpu.CompilerParams(use_tc_tiling_on_sc=False),
    )
    return f(instr2d, emb_table, pos_table, ln_gamma, ln_beta)


def kernel(instruction, emb_table, pos_table, ln_gamma, ln_beta):
    instr2d = instruction.astype(jnp.int32).reshape(NTOK // 128, 128)
    out = _run(instr2d, emb_table, pos_table, ln_gamma, ln_beta)
    return out.reshape(BATCH, SEQ, D)


# speedup vs baseline: 1.0210x; 1.0210x over previous
import jax
import jax.numpy as jnp
from jax import lax
from jax.experimental import pallas as pl
from jax.experimental.pallas import tpu as pltpu
from jax.experimental.pallas import tpu_sc as plsc

N_POS = 200
D = 64
BATCH = 4096
SEQ = 200
NTOK = BATCH * SEQ
NW = 32
CHUNK = 128
NCH = SEQ
NBUF = 8
UNROLL = 16


def _rsqrt(a):
    i = lax.bitcast_convert_type(a, jnp.int32)
    i = jnp.int32(0x5F3759DF) - (i >> 1)
    y = lax.bitcast_convert_type(i, jnp.float32)
    h = a * 0.5
    for _ in range(2):
        y = y * (1.5 - h * y * y)
    return y


_GATHER_DNUMS = lax.GatherDimensionNumbers(
    offset_dims=(), collapsed_slice_dims=(0,), start_index_map=(0,))


def _shuf(v, perm2d):
    return lax.gather(v, perm2d, _GATHER_DNUMS, slice_sizes=(1,),
                      mode=lax.GatherScatterMode.PROMISE_IN_BOUNDS)


def _lane_sum(v, perms):
    for perm2d in perms:
        v = v + _shuf(v, perm2d)
    return v


def _ln_row(rows_v, src_i, y_v, dst_i, pvec, gvec, bvec, perms):
    x = [rows_v[src_i, pl.ds(16 * j, 16)] + pvec[j] for j in range(4)]
    s = (x[0] + x[1]) + (x[2] + x[3])
    q = (x[0] * x[0] + x[1] * x[1]) + (x[2] * x[2] + x[3] * x[3])
    mean = _lane_sum(s, perms) * (1.0 / D)
    ex2 = _lane_sum(q, perms) * (1.0 / D)
    var = ex2 - mean * mean
    r = _rsqrt(var + 1e-5)
    for j in range(4):
        y = (x[j] - mean) * r * gvec[j] + bvec[j]
        y_v[dst_i, pl.ds(16 * j, 16)] = y


def _sc_body(instr_hbm, table_hbm, pos_hbm, gamma_hbm, beta_hbm, out_hbm,
             idx_all, rows_all, y0, y1, pos_v, g_v, b_v,
             gsems, ysem0, ysem1):
    wid = lax.axis_index("s") * 2 + lax.axis_index("c")
    b0 = wid * CHUNK
    ybuf = (y0, y1)
    ysem = (ysem0, ysem1)

    pltpu.sync_copy(instr_hbm.at[:, pl.ds(b0, CHUNK)], idx_all)
    pltpu.sync_copy(pos_hbm, pos_v)
    pltpu.sync_copy(gamma_hbm, g_v)
    pltpu.sync_copy(beta_hbm, b_v)
    gvec = [g_v[pl.ds(16 * j, 16)] for j in range(4)]
    bvec = [b_v[pl.ds(16 * j, 16)] for j in range(4)]
    lanes = lax.iota(jnp.int32, 16)
    perms = [(lanes ^ k).reshape(16, 1) for k in (8, 4, 2, 1)]

    def gather_start(g, b):
        pltpu.make_async_copy(
            table_hbm.at[idx_all.at[g]],
            rows_all.at[pl.ds(b * CHUNK, CHUNK)], gsems.at[b]).start()

    def gather_wait(g, b):
        pltpu.make_async_copy(
            table_hbm.at[idx_all.at[g]],
            rows_all.at[pl.ds(b * CHUNK, CHUNK)], gsems.at[b]).wait()

    def out_copy(g, b):
        return pltpu.make_async_copy(
            ybuf[b], out_hbm.at[pl.ds(b0, CHUNK), g], ysem[b])

    for b in range(NBUF):
        gather_start(b, b)

    def lap_body(t, carry):
        for b in range(NBUF):
            gc = NBUF * t + b
            gather_wait(gc, b)
            pvec = [pos_v[gc, pl.ds(16 * j, 16)] for j in range(4)]
            yb = b % 2

            @pl.when(gc >= 2)
            def _(gc=gc, yb=yb):
                out_copy(gc - 2, yb).wait()

            @plsc.parallel_loop(0, CHUNK, 1, unroll=UNROLL)
            def _row(i, b=b, yb=yb, pvec=pvec):
                _ln_row(rows_all, b * CHUNK + i, ybuf[yb], i,
                        pvec, gvec, bvec, perms)
            out_copy(gc, yb).start()

            @pl.when(gc + NBUF < NCH)
            def _(b=b, gc=gc):
                gather_start(gc + NBUF, b)
        return carry

    lax.fori_loop(0, NCH // NBUF, lap_body, 0)
    out_copy(NCH - 2, (NCH - 2) % 2).wait()
    out_copy(NCH - 1, (NCH - 1) % 2).wait()


@jax.jit
def _run(instr_t, emb_table, pos_table, ln_gamma, ln_beta):
    mesh = plsc.VectorSubcoreMesh(core_axis_name="c", subcore_axis_name="s")
    f = pl.kernel(
        _sc_body,
        mesh=mesh,
        out_type=jax.ShapeDtypeStruct((BATCH, SEQ, D), jnp.float32),
        scratch_types=[
            pltpu.VMEM((SEQ, CHUNK), jnp.int32),
            pltpu.VMEM((NBUF * CHUNK, D), jnp.float32),
            pltpu.VMEM((CHUNK, D), jnp.float32),
            pltpu.VMEM((CHUNK, D), jnp.float32),
            pltpu.VMEM((N_POS, D), jnp.float32),
            pltpu.VMEM((D,), jnp.float32),
            pltpu.VMEM((D,), jnp.float32),
            pltpu.SemaphoreType.DMA((NBUF,)),
            pltpu.SemaphoreType.DMA,
            pltpu.SemaphoreType.DMA,
        ],
        compiler_params=pltpu.CompilerParams(use_tc_tiling_on_sc=False),
    )
    return f(instr_t, emb_table, pos_table, ln_gamma, ln_beta)


def kernel(instruction, emb_table, pos_table, ln_gamma, ln_beta):
    instr_t = instruction.astype(jnp.int32).T
    return _run(instr_t, emb_table, pos_table, ln_gamma, ln_beta)
